# baseline (speedup 1.0000x reference)
import functools

import jax
import jax.numpy as jnp
from jax import lax
from jax.experimental import pallas as pl
from jax.experimental.pallas import tpu as pltpu
from jax.experimental.pallas import tpu_sc as plsc

VOCAB = 1000
ALIGNED = 896
TAILW = 128
N_TOK = 1024 * 50
NC, NS = 2, 16
NW = NC * NS
N_PIECE = 4
PIECE = N_TOK // N_PIECE
ROWS_PER_W = PIECE // NW
CHUNK = 40
N_CHUNKS = ROWS_PER_W // CHUNK
TAIL_OFFS = (0, 16, 32, 48, 64, 80, 88)
BT = 1280
TSTEPS = PIECE // BT


def _lse_body(tab_ref, lse_ref):
    x = tab_ref[...]
    m = jnp.max(x, axis=1, keepdims=True)
    s = jnp.sum(jnp.exp(x - m), axis=1, keepdims=True)
    lse_ref[...] = m + jnp.log(s)


def _row_lse(table):
    out = pl.pallas_call(
        _lse_body,
        out_shape=jax.ShapeDtypeStruct((VOCAB, 1), jnp.float32),
    )(table)
    return out.reshape(VOCAB)


@functools.partial(
    pl.kernel,
    out_type=jax.ShapeDtypeStruct((PIECE, VOCAB), jnp.float32),
    mesh=plsc.VectorSubcoreMesh(core_axis_name="c", subcore_axis_name="s"),
    compiler_params=pltpu.CompilerParams(needs_layout_passes=False),
    scratch_types=[
        pltpu.VMEM((2, CHUNK), jnp.int32),
        pltpu.VMEM((CHUNK, VOCAB), jnp.float32),
        pltpu.VMEM((CHUNK, VOCAB), jnp.float32),
        pltpu.VMEM((CHUNK, TAILW), jnp.float32),
        pltpu.VMEM((CHUNK, TAILW), jnp.float32),
        pltpu.SemaphoreType.DMA,
        pltpu.SemaphoreType.DMA,
        pltpu.SemaphoreType.DMA,
        pltpu.SemaphoreType.DMA,
    ],
)
def _sc_gather(taba_hbm, tabb_hbm, idx_hbm, out_hbm,
               idx_v, trim0_v, trim1_v, tail0_v, tail1_v,
               asem0, asem1, bsem0, bsem1):
    wid = lax.axis_index("s") * NC + lax.axis_index("c")
    w_base = wid * ROWS_PER_W
    trims = (trim0_v, trim1_v)
    tails = (tail0_v, tail1_v)
    asems = (asem0, asem1)
    bsems = (bsem0, bsem1)

    def start_gather(c, p):
        base = pl.multiple_of(w_base + c * CHUNK, CHUNK)
        pltpu.sync_copy(idx_hbm.at[pl.ds(base, CHUNK)], idx_v.at[p])
        pltpu.make_async_copy(
            taba_hbm.at[idx_v.at[p]],
            trims[p].at[:, pl.ds(0, ALIGNED)], asems[p]).start()
        pltpu.make_async_copy(
            tabb_hbm.at[idx_v.at[p]], tails[p], bsems[p]).start()

    def finish_chunk(c, p):
        base = pl.multiple_of(w_base + c * CHUNK, CHUNK)
        pltpu.make_async_copy(
            taba_hbm.at[idx_v.at[p]],
            trims[p].at[:, pl.ds(0, ALIGNED)], asems[p]).wait()
        pltpu.make_async_copy(
            tabb_hbm.at[idx_v.at[p]], tails[p], bsems[p]).wait()
        for r in range(CHUNK):
            for o in TAIL_OFFS:
                trims[p][r, pl.ds(ALIGNED + o, 16)] = tails[p][r, pl.ds(o, 16)]
        pltpu.sync_copy(trims[p], out_hbm.at[pl.ds(base, CHUNK)])

    start_gather(0, 0)
    start_gather(1, 1)

    def body(c, carry):
        @pl.when(c % 2 == 0)
        def _():
            finish_chunk(c, 0)

            @pl.when(c + 2 < N_CHUNKS)
            def _():
                start_gather(c + 2, 0)

        @pl.when(c % 2 == 1)
        def _():
            finish_chunk(c, 1)

            @pl.when(c + 2 < N_CHUNKS)
            def _():
                start_gather(c + 2, 1)

        return carry

    lax.fori_loop(0, N_CHUNKS, body, jnp.int32(0))


def _trans_body(g_ref, prev_ref, out_ref):
    del prev_ref
    out_ref[...] = g_ref[...].T


def _make_transpose(k, with_prev):
    koff = k * TSTEPS
    in_specs = [pl.BlockSpec((BT, VOCAB), lambda i: (i, 0))]
    if with_prev:
        in_specs.append(pl.BlockSpec(memory_space=pl.ANY))
        body = _trans_body
        aliases = {1: 0}
    else:
        def body(g_ref, out_ref):
            out_ref[...] = g_ref[...].T
        aliases = {}

    return pl.pallas_call(
        body,
        grid=(TSTEPS,),
        in_specs=in_specs,
        out_specs=pl.BlockSpec((VOCAB, BT), lambda i: (0, koff + i)),
        out_shape=jax.ShapeDtypeStruct((VOCAB, N_TOK), jnp.float32),
        input_output_aliases=aliases,
    )


TOK_PER_W = N_TOK // NW
N_VEC = TOK_PER_W // 16


@functools.partial(
    pl.kernel,
    out_type=jax.ShapeDtypeStruct((NW, 16), jnp.float32),
    mesh=plsc.VectorSubcoreMesh(core_axis_name="c", subcore_axis_name="s"),
    compiler_params=pltpu.CompilerParams(
        needs_layout_passes=False, use_tc_tiling_on_sc=False),
    scratch_types=[
        pltpu.VMEM((TOK_PER_W,), jnp.int32),
        pltpu.VMEM((TOK_PER_W,), jnp.int32),
        pltpu.VMEM((TOK_PER_W,), jnp.float32),
        pltpu.VMEM((TOK_PER_W,), jnp.float32),
        pltpu.VMEM((16,), jnp.float32),
        pltpu.SemaphoreType.DMA,
    ],
)
def _sc_loss(tabflat_hbm, idx_hbm, tgt_hbm, lse_hbm, dep_hbm, part_hbm,
             idx_v, fidx_v, tval_v, lval_v, acc_v, sem):
    del dep_hbm
    wid = lax.axis_index("s") * NC + lax.axis_index("c")
    base = pl.multiple_of(wid * TOK_PER_W, TOK_PER_W)
    pltpu.sync_copy(idx_hbm.at[pl.ds(base, TOK_PER_W)], idx_v)
    pltpu.sync_copy(tgt_hbm.at[pl.ds(base, TOK_PER_W)], fidx_v)

    def mk_fidx(k, carry):
        sl = pl.ds(k * 16, 16)
        fidx_v[sl] = idx_v[sl] * VOCAB + fidx_v[sl]
        return carry

    lax.fori_loop(0, N_VEC, mk_fidx, jnp.int32(0))
    pltpu.async_copy(tabflat_hbm.at[fidx_v], tval_v, sem).wait()
    pltpu.async_copy(lse_hbm.at[idx_v], lval_v, sem).wait()

    def accum(k, acc):
        sl = pl.ds(k * 16, 16)
        return acc + (lval_v[sl] - tval_v[sl])

    acc = lax.fori_loop(0, N_VEC, accum, jnp.zeros((16,), jnp.float32))
    acc_v[...] = acc
    pltpu.sync_copy(acc_v, part_hbm.at[wid])


def kernel(idx, target, table):
    lse = _row_lse(table)
    table_a = table[:, :ALIGNED]
    table_b = jnp.pad(table[:, ALIGNED:],
                      ((0, 0), (0, TAILW - (VOCAB - ALIGNED))))
    idx_flat = idx.reshape(N_TOK)
    buf = None
    last_piece = None
    for k in range(N_PIECE):
        last_piece = _sc_gather(
            table_a, table_b,
            lax.slice(idx_flat, (k * PIECE,), ((k + 1) * PIECE,)))
        if buf is None:
            buf = _make_transpose(0, False)(last_piece)
        else:
            buf = _make_transpose(k, True)(last_piece, buf)
    pieces = [last_piece]
    logits = buf.T
    dep = lax.slice(pieces[-1], (0, 0), (8, 8)).reshape(64)
    partials = _sc_loss(table.reshape(VOCAB * VOCAB), idx_flat, target, lse,
                        dep)
    loss = jnp.sum(partials) / N_TOK
    return (logits, loss)


# speedup vs baseline: 1.0097x; 1.0097x over previous
import functools

import jax
import jax.numpy as jnp
from jax import lax
from jax.experimental import pallas as pl
from jax.experimental.pallas import tpu as pltpu
from jax.experimental.pallas import tpu_sc as plsc

VOCAB = 1000
ALIGNED = 896
TAILW = 128
N_TOK = 1024 * 50
NC, NS = 2, 16
NW = NC * NS
N_PIECE = 4
PIECE = N_TOK // N_PIECE
ROWS_PER_W = PIECE // NW
CHUNK = 40
N_CHUNKS = ROWS_PER_W // CHUNK
TAIL_OFFS = (0, 16, 32, 48, 64, 80, 88)
BT = 2560
TSTEPS = PIECE // BT


def _lse_body(tab_ref, lse_ref):
    x = tab_ref[...]
    m = jnp.max(x, axis=1, keepdims=True)
    s = jnp.sum(jnp.exp(x - m), axis=1, keepdims=True)
    lse_ref[...] = m + jnp.log(s)


def _row_lse(table):
    out = pl.pallas_call(
        _lse_body,
        out_shape=jax.ShapeDtypeStruct((VOCAB, 1), jnp.float32),
    )(table)
    return out.reshape(VOCAB)


@functools.partial(
    pl.kernel,
    out_type=jax.ShapeDtypeStruct((PIECE, VOCAB), jnp.float32),
    mesh=plsc.VectorSubcoreMesh(core_axis_name="c", subcore_axis_name="s"),
    compiler_params=pltpu.CompilerParams(needs_layout_passes=False),
    scratch_types=[
        pltpu.VMEM((2, CHUNK), jnp.int32),
        pltpu.VMEM((CHUNK, VOCAB), jnp.float32),
        pltpu.VMEM((CHUNK, VOCAB), jnp.float32),
        pltpu.VMEM((CHUNK, TAILW), jnp.float32),
        pltpu.VMEM((CHUNK, TAILW), jnp.float32),
        pltpu.SemaphoreType.DMA,
        pltpu.SemaphoreType.DMA,
        pltpu.SemaphoreType.DMA,
        pltpu.SemaphoreType.DMA,
    ],
)
def _sc_gather(taba_hbm, tabb_hbm, idx_hbm, out_hbm,
               idx_v, trim0_v, trim1_v, tail0_v, tail1_v,
               asem0, asem1, bsem0, bsem1):
    wid = lax.axis_index("s") * NC + lax.axis_index("c")
    w_base = wid * ROWS_PER_W
    trims = (trim0_v, trim1_v)
    tails = (tail0_v, tail1_v)
    asems = (asem0, asem1)
    bsems = (bsem0, bsem1)

    def start_gather(c, p):
        base = pl.multiple_of(w_base + c * CHUNK, CHUNK)
        pltpu.sync_copy(idx_hbm.at[pl.ds(base, CHUNK)], idx_v.at[p])
        pltpu.make_async_copy(
            taba_hbm.at[idx_v.at[p]],
            trims[p].at[:, pl.ds(0, ALIGNED)], asems[p]).start()
        pltpu.make_async_copy(
            tabb_hbm.at[idx_v.at[p]], tails[p], bsems[p]).start()

    def finish_chunk(c, p):
        base = pl.multiple_of(w_base + c * CHUNK, CHUNK)
        pltpu.make_async_copy(
            taba_hbm.at[idx_v.at[p]],
            trims[p].at[:, pl.ds(0, ALIGNED)], asems[p]).wait()
        pltpu.make_async_copy(
            tabb_hbm.at[idx_v.at[p]], tails[p], bsems[p]).wait()
        for r in range(CHUNK):
            for o in TAIL_OFFS:
                trims[p][r, pl.ds(ALIGNED + o, 16)] = tails[p][r, pl.ds(o, 16)]
        pltpu.sync_copy(trims[p], out_hbm.at[pl.ds(base, CHUNK)])

    start_gather(0, 0)
    start_gather(1, 1)

    def body(c, carry):
        @pl.when(c % 2 == 0)
        def _():
            finish_chunk(c, 0)

            @pl.when(c + 2 < N_CHUNKS)
            def _():
                start_gather(c + 2, 0)

        @pl.when(c % 2 == 1)
        def _():
            finish_chunk(c, 1)

            @pl.when(c + 2 < N_CHUNKS)
            def _():
                start_gather(c + 2, 1)

        return carry

    lax.fori_loop(0, N_CHUNKS, body, jnp.int32(0))


def _trans_body(g_ref, prev_ref, out_ref):
    del prev_ref
    out_ref[...] = g_ref[...].T


def _make_transpose(k, with_prev):
    koff = k * TSTEPS
    in_specs = [pl.BlockSpec((BT, VOCAB), lambda i: (i, 0))]
    if with_prev:
        in_specs.append(pl.BlockSpec(memory_space=pl.ANY))
        body = _trans_body
        aliases = {1: 0}
    else:
        def body(g_ref, out_ref):
            out_ref[...] = g_ref[...].T
        aliases = {}

    return pl.pallas_call(
        body,
        grid=(TSTEPS,),
        in_specs=in_specs,
        out_specs=pl.BlockSpec((VOCAB, BT), lambda i: (0, koff + i)),
        out_shape=jax.ShapeDtypeStruct((VOCAB, N_TOK), jnp.float32),
        input_output_aliases=aliases,
    )


TOK_PER_W = N_TOK // NW
N_VEC = TOK_PER_W // 16


@functools.partial(
    pl.kernel,
    out_type=jax.ShapeDtypeStruct((NW, 16), jnp.float32),
    mesh=plsc.VectorSubcoreMesh(core_axis_name="c", subcore_axis_name="s"),
    compiler_params=pltpu.CompilerParams(
        needs_layout_passes=False, use_tc_tiling_on_sc=False),
    scratch_types=[
        pltpu.VMEM((TOK_PER_W,), jnp.int32),
        pltpu.VMEM((TOK_PER_W,), jnp.int32),
        pltpu.VMEM((TOK_PER_W,), jnp.float32),
        pltpu.VMEM((TOK_PER_W,), jnp.float32),
        pltpu.VMEM((16,), jnp.float32),
        pltpu.SemaphoreType.DMA,
    ],
)
def _sc_loss(tabflat_hbm, idx_hbm, tgt_hbm, lse_hbm, dep_hbm, part_hbm,
             idx_v, fidx_v, tval_v, lval_v, acc_v, sem):
    del dep_hbm
    wid = lax.axis_index("s") * NC + lax.axis_index("c")
    base = pl.multiple_of(wid * TOK_PER_W, TOK_PER_W)
    pltpu.sync_copy(idx_hbm.at[pl.ds(base, TOK_PER_W)], idx_v)
    pltpu.sync_copy(tgt_hbm.at[pl.ds(base, TOK_PER_W)], fidx_v)

    def mk_fidx(k, carry):
        sl = pl.ds(k * 16, 16)
        fidx_v[sl] = idx_v[sl] * VOCAB + fidx_v[sl]
        return carry

    lax.fori_loop(0, N_VEC, mk_fidx, jnp.int32(0))
    pltpu.async_copy(tabflat_hbm.at[fidx_v], tval_v, sem).wait()
    pltpu.async_copy(lse_hbm.at[idx_v], lval_v, sem).wait()

    def accum(k, acc):
        sl = pl.ds(k * 16, 16)
        return acc + (lval_v[sl] - tval_v[sl])

    acc = lax.fori_loop(0, N_VEC, accum, jnp.zeros((16,), jnp.float32))
    acc_v[...] = acc
    pltpu.sync_copy(acc_v, part_hbm.at[wid])


def kernel(idx, target, table):
    lse = _row_lse(table)
    table_a = table[:, :ALIGNED]
    table_b = jnp.pad(table[:, ALIGNED:],
                      ((0, 0), (0, TAILW - (VOCAB - ALIGNED))))
    idx_flat = idx.reshape(N_TOK)
    buf = None
    last_piece = None
    for k in range(N_PIECE):
        last_piece = _sc_gather(
            table_a, table_b,
            lax.slice(idx_flat, (k * PIECE,), ((k + 1) * PIECE,)))
        if buf is None:
            buf = _make_transpose(0, False)(last_piece)
        else:
            buf = _make_transpose(k, True)(last_piece, buf)
    pieces = [last_piece]
    logits = buf.T
    dep = lax.slice(pieces[-1], (0, 0), (8, 8)).reshape(64)
    partials = _sc_loss(table.reshape(VOCAB * VOCAB), idx_flat, target, lse,
                        dep)
    loss = jnp.sum(partials) / N_TOK
    return (logits, loss)
